# Initial kernel scaffold; baseline (speedup 1.0000x reference)
import jax
import jax.numpy as jnp
from jax.experimental import pallas as pl


def kernel(yhat, y, kls, lr):
    raise NotImplementedError("write your pallas kernel here")


# speedup vs baseline: 11.7756x; 11.7756x over previous
import functools

import jax
import jax.numpy as jnp
from jax import lax
from jax.experimental import pallas as pl
from jax.experimental.pallas import tpu as pltpu
from jax.experimental.pallas import tpu_sc as plsc

_K_FRAC = 0.01
_KAPPA = 1.0

_NC, _NS, _L = 2, 16, 16
_NW = _NC * _NS
_NB = 2048
_N = 4 * 2048 * 1024
_K = int(_N * _K_FRAC)
_PER_W = _N // _NW
_CHUNK = 4096
_NCHUNK = _PER_W // _CHUNK
_VPC = _CHUNK // _L

_mesh = plsc.VectorSubcoreMesh(core_axis_name="c", subcore_axis_name="s",
                               num_cores=_NC, num_subcores=_NS)
_cparams = pltpu.CompilerParams(needs_layout_passes=False)


def _worker_id():
    return lax.axis_index("s") * _NC + lax.axis_index("c")


def _zero_ref(ref, nwords):
    z = jnp.zeros((_L,), jnp.float32)

    def body(i, _):
        ref[pl.ds(i * _L, _L)] = z
        return 0

    lax.fori_loop(0, nwords // _L, body, 0)


def _lane_reduce(hist_ref, out_ref):
    def body(j, _):
        acc = jnp.zeros((_L,), jnp.float32)
        for l in range(_L):
            acc = acc + hist_ref[pl.ds(l * _NB + j * _L, _L)]
        out_ref[pl.ds(j * _L, _L)] = acc
        return 0

    lax.fori_loop(0, _NB // _L, body, 0)


def _accum_rows(hbm_ref, row_ref, acc_ref):
    _zero_ref(acc_ref, _NB)
    for w in range(_NW):
        pltpu.sync_copy(hbm_ref.at[w], row_ref)

        def body(j, _):
            s = pl.ds(j * _L, _L)
            acc_ref[s] = acc_ref[s] + row_ref[s]
            return 0

        lax.fori_loop(0, _NB // _L, body, 0)


def _find_bracket(cnt_ref, sum_ref, target):
    liota = lax.iota(jnp.int32, _L)

    def body(jj, carry):
        done, b, r, sumab, cb, sb, ccnt, csum = carry
        j = _NB // _L - 1 - jj
        vc = cnt_ref[pl.ds(j * _L, _L)]
        vs = sum_ref[pl.ds(j * _L, _L)]
        tot_c = jnp.sum(vc)
        cross = jnp.logical_and(jnp.logical_not(done), ccnt + tot_c >= target)
        srev = plsc.cumsum(lax.rev(vc, (0,))) + ccnt
        ssrev = plsc.cumsum(lax.rev(vs, (0,))) + csum
        maskv = srev >= target
        ffs = jnp.sum(jnp.where(maskv, jnp.int32(0), jnp.int32(1)))
        lane = jnp.int32(15) - ffs
        cntb = jnp.sum(jnp.where(liota == lane, vc, 0.0))
        sumb = jnp.sum(jnp.where(liota == lane, vs, 0.0))
        s_b = jnp.sum(jnp.where(liota == ffs, srev, 0.0))
        ss_b = jnp.sum(jnp.where(liota == ffs, ssrev, 0.0))
        nb = j * _L + lane
        nr = target - (s_b - cntb)
        nsum = ss_b - sumb
        done2 = jnp.logical_or(done, cross)
        return (done2,
                jnp.where(cross, nb, b),
                jnp.where(cross, nr, r),
                jnp.where(cross, nsum, sumab),
                jnp.where(cross, cntb, cb),
                jnp.where(cross, sumb, sb),
                ccnt + tot_c,
                csum + jnp.sum(vs))

    init = (jnp.bool_(False), jnp.int32(0), jnp.float32(0.0),
            jnp.float32(0.0), jnp.float32(0.0), jnp.float32(0.0),
            jnp.float32(0.0), jnp.float32(0.0))
    _, b, r, sumab, cb, sb, _, _ = lax.fori_loop(0, _NB // _L, body, init)
    return b, r, sumab, cb, sb


@functools.partial(
    pl.kernel,
    out_type=(jax.ShapeDtypeStruct((_NW, _NB), jnp.float32),
              jax.ShapeDtypeStruct((_NW, _NB), jnp.float32)),
    mesh=_mesh,
    compiler_params=_cparams,
    scratch_types=[
        pltpu.VMEM((_CHUNK,), jnp.float32),
        pltpu.VMEM((_CHUNK,), jnp.float32),
        pltpu.VMEM((_L * _NB,), jnp.float32),
        pltpu.VMEM((_L * _NB,), jnp.float32),
        pltpu.VMEM((_NB,), jnp.float32),
    ],
)
def _pass1(yhat_hbm, y_hbm, cnt_hbm, sum_hbm, abuf, bbuf, hcnt, hsum, orow):
    wid = _worker_id()
    base = wid * _PER_W
    _zero_ref(hcnt, _L * _NB)
    _zero_ref(hsum, _L * _NB)
    lane_base = lax.iota(jnp.int32, _L) * _NB
    ones = jnp.ones((_L,), jnp.float32)

    def chunk_body(c, _):
        off = base + c * _CHUNK
        pltpu.sync_copy(yhat_hbm.at[pl.ds(off, _CHUNK)], abuf)
        pltpu.sync_copy(y_hbm.at[pl.ds(off, _CHUNK)], bbuf)

        def vec_body(i, _):
            s = pl.ds(i * _L, _L)
            d = abuf[s] - bbuf[s]
            v = d * d
            u = lax.bitcast_convert_type(v, jnp.int32)
            idx = lax.shift_right_logical(u, 20) + lane_base
            plsc.addupdate_scatter(hcnt, [idx], ones)
            plsc.addupdate_scatter(hsum, [idx], v)
            return 0

        lax.fori_loop(0, _VPC, vec_body, 0)
        return 0

    lax.fori_loop(0, _NCHUNK, chunk_body, 0)
    _lane_reduce(hcnt, orow)
    pltpu.sync_copy(orow, cnt_hbm.at[wid])
    _lane_reduce(hsum, orow)
    pltpu.sync_copy(orow, sum_hbm.at[wid])


@functools.partial(
    pl.kernel,
    out_type=(jax.ShapeDtypeStruct((_NW, _NB), jnp.float32),
              jax.ShapeDtypeStruct((_NW, _NB), jnp.float32)),
    mesh=_mesh,
    compiler_params=_cparams,
    scratch_types=[
        pltpu.VMEM((_CHUNK,), jnp.float32),
        pltpu.VMEM((_CHUNK,), jnp.float32),
        pltpu.VMEM((_L * _NB,), jnp.float32),
        pltpu.VMEM((_L * _NB,), jnp.float32),
        pltpu.VMEM((_NB,), jnp.float32),
        pltpu.VMEM((_NB,), jnp.float32),
    ],
)
def _pass2(yhat_hbm, y_hbm, c1_hbm, s1_hbm, cnt_hbm, sum_hbm,
           abuf, bbuf, hcnt, hsum, orow, acc):
    wid = _worker_id()
    base = wid * _PER_W
    _accum_rows(c1_hbm, orow, acc)
    b1, _, _, _, _ = _find_bracket(acc, acc, jnp.float32(_K))

    _zero_ref(hcnt, _L * _NB)
    _zero_ref(hsum, _L * _NB)
    lane_base = lax.iota(jnp.int32, _L) * _NB
    ones = jnp.ones((_L,), jnp.float32)

    def chunk_body(c, _):
        off = base + c * _CHUNK
        pltpu.sync_copy(yhat_hbm.at[pl.ds(off, _CHUNK)], abuf)
        pltpu.sync_copy(y_hbm.at[pl.ds(off, _CHUNK)], bbuf)

        def vec_body(i, _):
            s = pl.ds(i * _L, _L)
            d = abuf[s] - bbuf[s]
            v = d * d
            u = lax.bitcast_convert_type(v, jnp.int32)
            hit = lax.shift_right_logical(u, 20) == b1
            idx = (lax.shift_right_logical(u, 9) & jnp.int32(_NB - 1)) + lane_base
            plsc.addupdate_scatter(hcnt, [idx], ones, mask=hit)
            plsc.addupdate_scatter(hsum, [idx], v, mask=hit)
            return 0

        lax.fori_loop(0, _VPC, vec_body, 0)
        return 0

    lax.fori_loop(0, _NCHUNK, chunk_body, 0)
    _lane_reduce(hcnt, orow)
    pltpu.sync_copy(orow, cnt_hbm.at[wid])
    _lane_reduce(hsum, orow)
    pltpu.sync_copy(orow, sum_hbm.at[wid])


@functools.partial(
    pl.kernel,
    out_type=jax.ShapeDtypeStruct((_L,), jnp.float32),
    mesh=_mesh,
    compiler_params=_cparams,
    scratch_types=[
        pltpu.VMEM((_NB,), jnp.float32),
        pltpu.VMEM((_NB,), jnp.float32),
        pltpu.VMEM((_NB,), jnp.float32),
        pltpu.VMEM((_NB,), jnp.float32),
        pltpu.VMEM((_NB,), jnp.float32),
        pltpu.VMEM((256,), jnp.float32),
        pltpu.VMEM((_L,), jnp.float32),
    ],
)
def _pass3(c1_hbm, s1_hbm, c2_hbm, s2_hbm, kls_hbm, out_hbm,
           row, c1, s1, c2, s2, kbuf, obuf):
    wid = _worker_id()

    @pl.when(wid == 0)
    def _():
        _accum_rows(c1_hbm, row, c1)
        _accum_rows(s1_hbm, row, s1)
        _accum_rows(c2_hbm, row, c2)
        _accum_rows(s2_hbm, row, s2)
        _, r1, above1, _, _ = _find_bracket(c1, s1, jnp.float32(_K))
        _, r2, above2, cb2, sb2 = _find_bracket(c2, s2, r1)
        liota = lax.iota(jnp.int32, _L)
        xv = jnp.where(liota == 0, jnp.maximum(cb2, 1.0), 1.0)
        iv = lax.bitcast_convert_type(xv, jnp.int32)
        rv = lax.bitcast_convert_type(jnp.int32(0x7EF311C3) - iv, jnp.float32)
        for _unused in range(3):
            rv = rv * (jnp.float32(2.0) - xv * rv)
        inv_cb2 = jnp.sum(jnp.where(liota == 0, rv, 0.0))
        topk_sum = above1 + above2 + r2 * sb2 * inv_cb2

        pltpu.sync_copy(kls_hbm, kbuf)
        ksum = jnp.float32(0.0)

        def kbody(i, acc):
            return acc + jnp.sum(kbuf[pl.ds(i * _L, _L)])

        ksum = lax.fori_loop(0, 256 // _L, kbody, ksum)

        rec_loss = topk_sum * jnp.float32(1.0 / _K_FRAC / 4.0)
        loss = rec_loss - jnp.float32(_KAPPA) + ksum
        obuf[pl.ds(0, _L)] = jnp.where(liota == 0, loss, 0.0)
        pltpu.sync_copy(obuf, out_hbm)


def kernel(yhat, y, kls, lr):
    del lr
    yf = jnp.reshape(yhat, (_N,))
    xf = jnp.reshape(y, (_N,))
    c1, s1 = _pass1(yf, xf)
    c2, s2 = _pass2(yf, xf, c1, s1)
    out = _pass3(c1, s1, c2, s2, jnp.reshape(kls, (256,)))
    return out[0]


# speedup vs baseline: 20.4720x; 1.7385x over previous
import functools

import jax
import jax.numpy as jnp
from jax import lax
from jax.experimental import pallas as pl
from jax.experimental.pallas import tpu as pltpu
from jax.experimental.pallas import tpu_sc as plsc

_K_FRAC = 0.01
_KAPPA = 1.0

_NC, _NS, _L = 2, 16, 16
_NW = _NC * _NS
_NB = 2048
_N = 4 * 2048 * 1024
_K = int(_N * _K_FRAC)
_PER_W = _N // _NW
_CHUNK = 4096
_NCHUNK = _PER_W // _CHUNK
_VPC = _CHUNK // _L
_UNROLL = 4

_mesh = plsc.VectorSubcoreMesh(core_axis_name="c", subcore_axis_name="s",
                               num_cores=_NC, num_subcores=_NS)
_cparams = pltpu.CompilerParams(needs_layout_passes=False)


def _worker_id():
    return lax.axis_index("s") * _NC + lax.axis_index("c")


def _zero_2d(ref):
    z = jnp.zeros((_L,), jnp.float32)

    def body(j, _):
        s = pl.ds(j * _L, _L)
        for w in range(_NS):
            ref[w, s] = z
        return 0

    lax.fori_loop(0, _NB // _L, body, 0)


def _merge_halves(hbm_ref, stage_ref, acc_ref):
    z = jnp.zeros((_L,), jnp.float32)

    def zbody(j, _):
        acc_ref[pl.ds(j * _L, _L)] = z
        return 0

    lax.fori_loop(0, _NB // _L, zbody, 0)
    for h in range(_NW // _NS):
        pltpu.sync_copy(hbm_ref.at[pl.ds(h * _NS, _NS)], stage_ref)

        def rbody(j, _):
            s = pl.ds(j * _L, _L)
            a = acc_ref[s]
            for w in range(_NS):
                a = a + stage_ref[w, s]
            acc_ref[s] = a
            return 0

        lax.fori_loop(0, _NB // _L, rbody, 0)


def _lane_reduce(hist_ref, out_ref):
    def body(j, _):
        s = pl.ds(j * _L, _L)
        acc = jnp.zeros((_L,), jnp.float32)
        for w in range(_NS):
            acc = acc + hist_ref[w, s]
        out_ref[pl.ds(j * _L, _L)] = acc
        return 0

    lax.fori_loop(0, _NB // _L, body, 0)


def _find_bracket(cnt_ref, sum_ref, target):
    liota = lax.iota(jnp.int32, _L)

    def body(jj, carry):
        done, b, r, sumab, cb, sb, ccnt, csum = carry
        j = _NB // _L - 1 - jj
        vc = cnt_ref[pl.ds(j * _L, _L)]
        vs = sum_ref[pl.ds(j * _L, _L)]
        tot_c = jnp.sum(vc)
        cross = jnp.logical_and(jnp.logical_not(done), ccnt + tot_c >= target)
        srev = plsc.cumsum(lax.rev(vc, (0,))) + ccnt
        ssrev = plsc.cumsum(lax.rev(vs, (0,))) + csum
        maskv = srev >= target
        ffs = jnp.sum(jnp.where(maskv, jnp.int32(0), jnp.int32(1)))
        lane = jnp.int32(15) - ffs
        cntb = jnp.sum(jnp.where(liota == lane, vc, 0.0))
        sumb = jnp.sum(jnp.where(liota == lane, vs, 0.0))
        s_b = jnp.sum(jnp.where(liota == ffs, srev, 0.0))
        ss_b = jnp.sum(jnp.where(liota == ffs, ssrev, 0.0))
        nb = j * _L + lane
        nr = target - (s_b - cntb)
        nsum = ss_b - sumb
        done2 = jnp.logical_or(done, cross)
        return (done2,
                jnp.where(cross, nb, b),
                jnp.where(cross, nr, r),
                jnp.where(cross, nsum, sumab),
                jnp.where(cross, cntb, cb),
                jnp.where(cross, sumb, sb),
                ccnt + tot_c,
                csum + jnp.sum(vs))

    init = (jnp.bool_(False), jnp.int32(0), jnp.float32(0.0),
            jnp.float32(0.0), jnp.float32(0.0), jnp.float32(0.0),
            jnp.float32(0.0), jnp.float32(0.0))
    _, b, r, sumab, cb, sb, _, _ = lax.fori_loop(0, _NB // _L, body, init)
    return b, r, sumab, cb, sb


def _stream(yhat_hbm, y_hbm, base, bufs, sems, process_chunk):
    (a0, b0), (a1, b1) = bufs
    s0, s1 = sems
    dummy = yhat_hbm.at[pl.ds(0, _CHUNK)]

    def fill(buf_a, buf_b, sem, c):
        off = base + c * _CHUNK
        pltpu.async_copy(yhat_hbm.at[pl.ds(off, _CHUNK)], buf_a, sem)
        pltpu.async_copy(y_hbm.at[pl.ds(off, _CHUNK)], buf_b, sem)

    def drain(buf_a, buf_b, sem):
        pltpu.make_async_copy(dummy, buf_a, sem).wait()
        pltpu.make_async_copy(dummy, buf_b, sem).wait()

    fill(a0, b0, s0, 0)
    fill(a1, b1, s1, 1)

    def outer(t, carry):
        c = 2 * t
        drain(a0, b0, s0)
        carry = process_chunk(a0, b0, carry)

        @pl.when(c + 2 < _NCHUNK)
        def _():
            fill(a0, b0, s0, c + 2)

        drain(a1, b1, s1)
        carry = process_chunk(a1, b1, carry)

        @pl.when(c + 3 < _NCHUNK)
        def _():
            fill(a1, b1, s1, c + 3)

        return carry

    return lax.fori_loop(0, _NCHUNK // 2, outer, 0.0 * jnp.zeros((_L,), jnp.float32))


@functools.partial(
    pl.kernel,
    out_type=jax.ShapeDtypeStruct((_NW, _NB), jnp.float32),
    mesh=_mesh,
    compiler_params=_cparams,
    scratch_types=[
        pltpu.VMEM((_CHUNK,), jnp.float32),
        pltpu.VMEM((_CHUNK,), jnp.float32),
        pltpu.VMEM((_CHUNK,), jnp.float32),
        pltpu.VMEM((_CHUNK,), jnp.float32),
        pltpu.VMEM((_NS, _NB), jnp.float32),
        pltpu.VMEM((_NB,), jnp.float32),
        pltpu.SemaphoreType.DMA,
        pltpu.SemaphoreType.DMA,
    ],
)
def _pass1(yhat_hbm, y_hbm, cnt_hbm, a0, b0, a1, b1, hcnt, orow, s0, s1):
    wid = _worker_id()
    base = wid * _PER_W
    _zero_2d(hcnt)
    liota = lax.iota(jnp.int32, _L)
    ones = jnp.ones((_L,), jnp.float32)

    def process_chunk(abuf, bbuf, carry):
        def vbody(i, _):
            for uu in range(_UNROLL):
                s = pl.ds((i * _UNROLL + uu) * _L, _L)
                d = abuf[s] - bbuf[s]
                v = d * d
                u = lax.bitcast_convert_type(v, jnp.int32)
                idx = lax.shift_right_logical(u, 20)
                plsc.addupdate_scatter(hcnt, [liota, idx], ones)
            return 0

        lax.fori_loop(0, _VPC // _UNROLL, vbody, 0)
        return carry

    _stream(yhat_hbm, y_hbm, base, ((a0, b0), (a1, b1)), (s0, s1),
            process_chunk)
    _lane_reduce(hcnt, orow)
    pltpu.sync_copy(orow, cnt_hbm.at[wid])


@functools.partial(
    pl.kernel,
    out_type=(jax.ShapeDtypeStruct((_NW, _NB), jnp.float32),
              jax.ShapeDtypeStruct((_NW, _NB), jnp.float32),
              jax.ShapeDtypeStruct((_NW, _L), jnp.float32)),
    mesh=_mesh,
    compiler_params=_cparams,
    scratch_types=[
        pltpu.VMEM((_CHUNK,), jnp.float32),
        pltpu.VMEM((_CHUNK,), jnp.float32),
        pltpu.VMEM((_CHUNK,), jnp.float32),
        pltpu.VMEM((_CHUNK,), jnp.float32),
        pltpu.VMEM((_NS, _NB), jnp.float32),
        pltpu.VMEM((_NS, _NB), jnp.float32),
        pltpu.VMEM((_NB,), jnp.float32),
        pltpu.VMEM((_L,), jnp.float32),
        pltpu.SemaphoreType.DMA,
        pltpu.SemaphoreType.DMA,
    ],
)
def _pass2(yhat_hbm, y_hbm, c1_hbm, cnt_hbm, sum_hbm, abv_hbm,
           a0, b0, a1, b1, hcnt, hsum, orow, vbuf, s0, s1):
    wid = _worker_id()
    base = wid * _PER_W
    _merge_halves(c1_hbm, hcnt, orow)
    bin1, _, _, _, _ = _find_bracket(orow, orow, jnp.float32(_K))

    _zero_2d(hcnt)
    _zero_2d(hsum)
    liota = lax.iota(jnp.int32, _L)
    ones = jnp.ones((_L,), jnp.float32)
    zf = jnp.zeros((_L,), jnp.float32)

    def process_chunk(abuf, bbuf, acc):
        def vbody(i, acc):
            for uu in range(_UNROLL):
                s = pl.ds((i * _UNROLL + uu) * _L, _L)
                d = abuf[s] - bbuf[s]
                v = d * d
                u = lax.bitcast_convert_type(v, jnp.int32)
                top = lax.shift_right_logical(u, 20)
                hit = top == bin1
                acc = acc + jnp.where(top > bin1, v, zf)
                idx = lax.shift_right_logical(u, 9) & jnp.int32(_NB - 1)
                plsc.addupdate_scatter(hcnt, [liota, idx], ones, mask=hit)
                plsc.addupdate_scatter(hsum, [liota, idx], v, mask=hit)
            return acc

        return lax.fori_loop(0, _VPC // _UNROLL, vbody, acc)

    above = _stream(yhat_hbm, y_hbm, base, ((a0, b0), (a1, b1)), (s0, s1),
                    process_chunk)
    _lane_reduce(hcnt, orow)
    pltpu.sync_copy(orow, cnt_hbm.at[wid])
    _lane_reduce(hsum, orow)
    pltpu.sync_copy(orow, sum_hbm.at[wid])
    vbuf[pl.ds(0, _L)] = above
    pltpu.sync_copy(vbuf, abv_hbm.at[wid])


@functools.partial(
    pl.kernel,
    out_type=jax.ShapeDtypeStruct((_L,), jnp.float32),
    mesh=_mesh,
    compiler_params=_cparams,
    scratch_types=[
        pltpu.VMEM((_NS, _NB), jnp.float32),
        pltpu.VMEM((_NB,), jnp.float32),
        pltpu.VMEM((_NB,), jnp.float32),
        pltpu.VMEM((_NB,), jnp.float32),
        pltpu.VMEM((_NW, _L), jnp.float32),
        pltpu.VMEM((256,), jnp.float32),
        pltpu.VMEM((_L,), jnp.float32),
    ],
)
def _pass3(c1_hbm, c2_hbm, s2_hbm, abv_hbm, kls_hbm, out_hbm,
           stage, c1, c2, s2, abuf, kbuf, obuf):
    wid = _worker_id()

    @pl.when(wid == 0)
    def _():
        _merge_halves(c1_hbm, stage, c1)
        _merge_halves(c2_hbm, stage, c2)
        _merge_halves(s2_hbm, stage, s2)
        _, r1, _, _, _ = _find_bracket(c1, c1, jnp.float32(_K))
        _, r2, above2, cb2, sb2 = _find_bracket(c2, s2, r1)

        pltpu.sync_copy(abv_hbm, abuf)
        above1 = jnp.float32(0.0)

        def abody(w, acc):
            return acc + jnp.sum(abuf[w, pl.ds(0, _L)])

        above1 = lax.fori_loop(0, _NW, abody, above1)

        liota = lax.iota(jnp.int32, _L)
        xv = jnp.where(liota == 0, jnp.maximum(cb2, 1.0), 1.0)
        iv = lax.bitcast_convert_type(xv, jnp.int32)
        rv = lax.bitcast_convert_type(jnp.int32(0x7EF311C3) - iv, jnp.float32)
        for _unused in range(3):
            rv = rv * (jnp.float32(2.0) - xv * rv)
        inv_cb2 = jnp.sum(jnp.where(liota == 0, rv, 0.0))
        topk_sum = above1 + above2 + r2 * sb2 * inv_cb2

        pltpu.sync_copy(kls_hbm, kbuf)
        ksum = jnp.float32(0.0)

        def kbody(i, acc):
            return acc + jnp.sum(kbuf[pl.ds(i * _L, _L)])

        ksum = lax.fori_loop(0, 256 // _L, kbody, ksum)

        rec_loss = topk_sum * jnp.float32(1.0 / _K_FRAC / 4.0)
        loss = rec_loss - jnp.float32(_KAPPA) + ksum
        obuf[pl.ds(0, _L)] = jnp.where(liota == 0, loss, 0.0)
        pltpu.sync_copy(obuf, out_hbm)


def kernel(yhat, y, kls, lr):
    del lr
    yf = jnp.reshape(yhat, (_N,))
    xf = jnp.reshape(y, (_N,))
    c1 = _pass1(yf, xf)
    c2, s2, abv = _pass2(yf, xf, c1)
    out = _pass3(c1, c2, s2, abv, jnp.reshape(kls, (256,)))
    return out[0]
